# Initial kernel scaffold; baseline (speedup 1.0000x reference)
import jax
import jax.numpy as jnp
from jax.experimental import pallas as pl


def kernel(q_pts, s_pts, s_feats, neighb_inds, W1, b1, gamma, beta, W2, b2):
    raise NotImplementedError("write your pallas kernel here")


# speedup vs baseline: 1.1408x; 1.1408x over previous
import functools

import jax
import jax.numpy as jnp
import numpy as np
from jax import lax
from jax.experimental import pallas as pl
from jax.experimental.pallas import tpu as pltpu
from jax.experimental.pallas import tpu_sc as plsc

_N = 10000
_H = 16
_C = 256
_K = 15
_CPG = 8
_G = _C // _CPG
_CR = 64
_SIGMA = 1.2
_BN_EPS = 1e-5

_D = 384
_B = _N * _H
_NW = 32
_RPW = _B // _NW
_CHUNK = 40
_NCHUNK = _RPW // _CHUNK

_BN = 200
_NB = _N // _BN
_R = _BN * _H


def _kp_const():
    rng = np.random.RandomState(42)
    pts = rng.randn(_K, 3)
    pts = pts / (np.linalg.norm(pts, axis=1, keepdims=True) + 1e-9)
    r = rng.rand(_K, 1) ** (1.0 / 3.0)
    pts = pts * r * 1.2
    pts[0, :] = 0.0
    return pts.astype(np.float32)


def _sc_gather(table, idx):
    mesh = plsc.VectorSubcoreMesh(core_axis_name="c", subcore_axis_name="s")

    @functools.partial(
        pl.kernel,
        mesh=mesh,
        out_type=jax.ShapeDtypeStruct((_B, _D), jnp.float32),
        scratch_types=[
            pltpu.VMEM((_CHUNK,), jnp.int32),
            pltpu.VMEM((_CHUNK, _D), jnp.float32),
            pltpu.SemaphoreType.DMA,
        ],
    )
    def k(table_hbm, idx_hbm, out_hbm, idx_v, rows_v, sem):
        wid = lax.axis_index("s") * 2 + lax.axis_index("c")
        base = wid * _RPW

        def body(c, carry):
            off = base + c * _CHUNK
            pltpu.sync_copy(idx_hbm.at[pl.ds(off, _CHUNK)], idx_v)
            pltpu.async_copy(table_hbm.at[idx_v], rows_v, sem).wait()
            pltpu.sync_copy(rows_v, out_hbm.at[pl.ds(off, _CHUNK)])
            return carry

        lax.fori_loop(0, _NCHUNK, body, 0)

    return k(table, idx)


def _stats_body(x_ref, w1_ref, b1_ref, s1_ref, s2_ref):
    h = jnp.dot(x_ref[...], w1_ref[...],
                preferred_element_type=jnp.float32) + b1_ref[...]

    @pl.when(pl.program_id(0) == 0)
    def _():
        s1_ref[...] = jnp.zeros_like(s1_ref)
        s2_ref[...] = jnp.zeros_like(s2_ref)

    s1_ref[...] += jnp.sum(h, axis=0, keepdims=True)
    s2_ref[...] += jnp.sum(h * h, axis=0, keepdims=True)


def _tc_stats(s_feats, W1, b1r):
    return pl.pallas_call(
        _stats_body,
        grid=(_NB,),
        in_specs=[
            pl.BlockSpec((_BN, _C), lambda i: (i, 0)),
            pl.BlockSpec((_C, _CR), lambda i: (0, 0)),
            pl.BlockSpec((1, _CR), lambda i: (0, 0)),
        ],
        out_specs=[
            pl.BlockSpec((1, _CR), lambda i: (0, 0)),
            pl.BlockSpec((1, _CR), lambda i: (0, 0)),
        ],
        out_shape=[
            jax.ShapeDtypeStruct((1, _CR), jnp.float32),
            jax.ShapeDtypeStruct((1, _CR), jnp.float32),
        ],
        compiler_params=pltpu.CompilerParams(
            dimension_semantics=("arbitrary",)),
    )(s_feats, W1, b1r)


def _rep_rows(a, m):
    return jnp.broadcast_to(a[:, None, :], (_BN, _H, m)).reshape(_R, m)


def _main_body(x_ref, q_ref, g_ref, s1_ref, s2_ref, w1_ref, b1_ref,
               gam_ref, bet_ref, w2_ref, b2_ref, kp_ref, o_ref):
    x = x_ref[...]
    h = jnp.dot(x, w1_ref[...], preferred_element_type=jnp.float32)
    h = h + b1_ref[...]
    mean = s1_ref[...] * (1.0 / _N)
    var = s2_ref[...] * (1.0 / _N) - mean * mean
    inv = lax.rsqrt(var + _BN_EPS)
    h = (h - mean) * (inv * gam_ref[...]) + bet_ref[...]
    h = jnp.where(h >= 0.0, h, 0.1 * h)
    cw = jnp.dot(h, w2_ref[...], preferred_element_type=jnp.float32)
    cw = cw + b2_ref[...]

    gb = g_ref[...]
    feats = gb[:, :_C]
    q = q_ref[...]
    kp = kp_ref[...]

    ax = gb[:, _C:_C + 1] - _rep_rows(q[:, 0:1], 1)
    ay = gb[:, _C + 1:_C + 2] - _rep_rows(q[:, 1:2], 1)
    az = gb[:, _C + 2:_C + 3] - _rep_rows(q[:, 2:3], 1)
    d2 = ((ax - kp[0:1, :]) ** 2 + (ay - kp[1:2, :]) ** 2
          + (az - kp[2:3, :]) ** 2)
    nw = jnp.maximum(1.0 - jnp.sqrt(d2) * (1.0 / _SIGMA), 0.0)

    agg = jnp.zeros((_R, _G), jnp.float32)
    for k in range(_K):
        cwk = _rep_rows(cw[:, k * _G:(k + 1) * _G], _G)
        agg = agg + nw[:, k:k + 1] * cwk

    g_ids = lax.broadcasted_iota(jnp.int32, (_G, _C), 0)
    c_ids = lax.broadcasted_iota(jnp.int32, (_G, _C), 1)
    expand = (c_ids // _CPG == g_ids).astype(jnp.float32)
    agg_exp = jnp.dot(agg, expand, preferred_element_type=jnp.float32)

    prod = agg_exp * feats
    o_ref[...] = jnp.sum(prod.reshape(_BN, _H, _C), axis=1)


def _tc_main(s_feats, q_pts, gathered, s1, s2, W1, b1r, gamr, betr,
             W2, b2r, kp3):
    return pl.pallas_call(
        _main_body,
        grid=(_NB,),
        in_specs=[
            pl.BlockSpec((_BN, _C), lambda i: (i, 0)),
            pl.BlockSpec((_BN, 3), lambda i: (i, 0)),
            pl.BlockSpec((_R, _D), lambda i: (i, 0)),
            pl.BlockSpec((1, _CR), lambda i: (0, 0)),
            pl.BlockSpec((1, _CR), lambda i: (0, 0)),
            pl.BlockSpec((_C, _CR), lambda i: (0, 0)),
            pl.BlockSpec((1, _CR), lambda i: (0, 0)),
            pl.BlockSpec((1, _CR), lambda i: (0, 0)),
            pl.BlockSpec((1, _CR), lambda i: (0, 0)),
            pl.BlockSpec((_CR, _K * _G), lambda i: (0, 0)),
            pl.BlockSpec((1, _K * _G), lambda i: (0, 0)),
            pl.BlockSpec((3, _K), lambda i: (0, 0)),
        ],
        out_specs=pl.BlockSpec((_BN, _C), lambda i: (i, 0)),
        out_shape=jax.ShapeDtypeStruct((_N, _C), jnp.float32),
        compiler_params=pltpu.CompilerParams(
            dimension_semantics=("arbitrary",)),
    )(s_feats, q_pts, gathered, s1, s2, W1, b1r, gamr, betr, W2, b2r, kp3)


def kernel(q_pts, s_pts, s_feats, neighb_inds, W1, b1, gamma, beta, W2, b2):
    table = jnp.concatenate(
        [s_feats, s_pts, jnp.zeros((_N, _D - _C - 3), jnp.float32)], axis=1)
    idx = neighb_inds.reshape(-1).astype(jnp.int32)
    gathered = _sc_gather(table, idx)
    b1r = b1.reshape(1, _CR)
    s1, s2 = _tc_stats(s_feats, W1, b1r)
    kp3 = jnp.asarray(_kp_const().T)
    out = _tc_main(s_feats, q_pts, gathered, s1, s2, W1, b1r,
                   gamma.reshape(1, _CR), beta.reshape(1, _CR),
                   W2, b2.reshape(1, _K * _G), kp3)
    return out


# speedup vs baseline: 1.8376x; 1.6108x over previous
import functools

import jax
import jax.numpy as jnp
import numpy as np
from jax import lax
from jax.experimental import pallas as pl
from jax.experimental.pallas import tpu as pltpu
from jax.experimental.pallas import tpu_sc as plsc

_N = 10000
_H = 16
_C = 256
_K = 15
_CPG = 8
_G = _C // _CPG
_CR = 64
_SIGMA = 1.2
_BN_EPS = 1e-5

_D = 384
_B = _N * _H
_NW = 32
_RPW = _B // _NW
_CHUNK = 40
_NCHUNK = _RPW // _CHUNK

_BN = 200
_NB = _N // _BN
_R = _BN * _H


def _kp_const():
    rng = np.random.RandomState(42)
    pts = rng.randn(_K, 3)
    pts = pts / (np.linalg.norm(pts, axis=1, keepdims=True) + 1e-9)
    r = rng.rand(_K, 1) ** (1.0 / 3.0)
    pts = pts * r * 1.2
    pts[0, :] = 0.0
    return pts.astype(np.float32)


def _sc_gather(table, idx):
    mesh = plsc.VectorSubcoreMesh(core_axis_name="c", subcore_axis_name="s")

    @functools.partial(
        pl.kernel,
        mesh=mesh,
        out_type=jax.ShapeDtypeStruct((_B, _D), jnp.float32),
        scratch_types=[
            pltpu.VMEM((_CHUNK,), jnp.int32),
            pltpu.VMEM((_CHUNK, _D), jnp.float32),
            pltpu.SemaphoreType.DMA,
        ],
    )
    def k(table_hbm, idx_hbm, out_hbm, idx_v, rows_v, sem):
        wid = lax.axis_index("s") * 2 + lax.axis_index("c")
        base = wid * _RPW

        def body(c, carry):
            off = base + c * _CHUNK
            pltpu.sync_copy(idx_hbm.at[pl.ds(off, _CHUNK)], idx_v)
            pltpu.async_copy(table_hbm.at[idx_v], rows_v, sem).wait()
            pltpu.sync_copy(rows_v, out_hbm.at[pl.ds(off, _CHUNK)])
            return carry

        lax.fori_loop(0, _NCHUNK, body, 0)

    return k(table, idx)


def _stats_body(x_ref, w1_ref, b1_ref, s1_ref, s2_ref):
    h = jnp.dot(x_ref[...], w1_ref[...],
                preferred_element_type=jnp.float32) + b1_ref[...]

    @pl.when(pl.program_id(0) == 0)
    def _():
        s1_ref[...] = jnp.zeros_like(s1_ref)
        s2_ref[...] = jnp.zeros_like(s2_ref)

    s1_ref[...] += jnp.sum(h, axis=0, keepdims=True)
    s2_ref[...] += jnp.sum(h * h, axis=0, keepdims=True)


def _tc_stats(s_feats, W1, b1r):
    return pl.pallas_call(
        _stats_body,
        grid=(_NB,),
        in_specs=[
            pl.BlockSpec((_BN, _C), lambda i: (i, 0)),
            pl.BlockSpec((_C, _CR), lambda i: (0, 0)),
            pl.BlockSpec((1, _CR), lambda i: (0, 0)),
        ],
        out_specs=[
            pl.BlockSpec((1, _CR), lambda i: (0, 0)),
            pl.BlockSpec((1, _CR), lambda i: (0, 0)),
        ],
        out_shape=[
            jax.ShapeDtypeStruct((1, _CR), jnp.float32),
            jax.ShapeDtypeStruct((1, _CR), jnp.float32),
        ],
        compiler_params=pltpu.CompilerParams(
            dimension_semantics=("arbitrary",)),
    )(s_feats, W1, b1r)


def _rep_rows(a, m):
    return jnp.broadcast_to(a[:, None, :], (_BN, _H, m)).reshape(_R, m)


def _main_body(x_ref, q_ref, g_ref, s1_ref, s2_ref, w1_ref, b1_ref,
               gam_ref, bet_ref, w2_ref, b2_ref, kp_ref, o_ref):
    x = x_ref[...]
    h = jnp.dot(x, w1_ref[...], preferred_element_type=jnp.float32)
    h = h + b1_ref[...]
    mean = s1_ref[...] * (1.0 / _N)
    var = s2_ref[...] * (1.0 / _N) - mean * mean
    inv = lax.rsqrt(var + _BN_EPS)
    h = (h - mean) * (inv * gam_ref[...]) + bet_ref[...]
    h = jnp.where(h >= 0.0, h, 0.1 * h)
    cw = jnp.dot(h, w2_ref[...], preferred_element_type=jnp.float32)
    cw = cw + b2_ref[...]

    gb = g_ref[...]
    feats = gb[:, :_C]
    q = q_ref[...]
    kpx = kp_ref[...]

    ax = gb[:, _C:_C + 1] - _rep_rows(q[:, 0:1], 1)
    ay = gb[:, _C + 1:_C + 2] - _rep_rows(q[:, 1:2], 1)
    az = gb[:, _C + 2:_C + 3] - _rep_rows(q[:, 2:3], 1)
    d2 = ((ax - kpx[0:1, :]) ** 2 + (ay - kpx[1:2, :]) ** 2
          + (az - kpx[2:3, :]) ** 2)
    nwx = jnp.maximum(1.0 - jnp.sqrt(d2) * (1.0 / _SIGMA), 0.0)

    cwr = _rep_rows(cw, _K * _G)
    prod480 = nwx * cwr

    kg_ids = lax.broadcasted_iota(jnp.int32, (_K * _G, _G), 0)
    gg_ids = lax.broadcasted_iota(jnp.int32, (_K * _G, _G), 1)
    et = (kg_ids % _G == gg_ids).astype(jnp.float32)
    agg = jnp.dot(prod480, et, preferred_element_type=jnp.float32)

    g_ids = lax.broadcasted_iota(jnp.int32, (_G, _C), 0)
    c_ids = lax.broadcasted_iota(jnp.int32, (_G, _C), 1)
    expand = (c_ids // _CPG == g_ids).astype(jnp.float32)
    agg_exp = jnp.dot(agg, expand, preferred_element_type=jnp.float32)

    prod = agg_exp * feats
    o_ref[...] = jnp.sum(prod.reshape(_BN, _H, _C), axis=1)


def _tc_main(s_feats, q_pts, gathered, s1, s2, W1, b1r, gamr, betr,
             W2, b2r, kp3):
    return pl.pallas_call(
        _main_body,
        grid=(_NB,),
        in_specs=[
            pl.BlockSpec((_BN, _C), lambda i: (i, 0)),
            pl.BlockSpec((_BN, 3), lambda i: (i, 0)),
            pl.BlockSpec((_R, _D), lambda i: (i, 0)),
            pl.BlockSpec((1, _CR), lambda i: (0, 0)),
            pl.BlockSpec((1, _CR), lambda i: (0, 0)),
            pl.BlockSpec((_C, _CR), lambda i: (0, 0)),
            pl.BlockSpec((1, _CR), lambda i: (0, 0)),
            pl.BlockSpec((1, _CR), lambda i: (0, 0)),
            pl.BlockSpec((1, _CR), lambda i: (0, 0)),
            pl.BlockSpec((_CR, _K * _G), lambda i: (0, 0)),
            pl.BlockSpec((1, _K * _G), lambda i: (0, 0)),
            pl.BlockSpec((3, _K * _G), lambda i: (0, 0)),
        ],
        out_specs=pl.BlockSpec((_BN, _C), lambda i: (i, 0)),
        out_shape=jax.ShapeDtypeStruct((_N, _C), jnp.float32),
        compiler_params=pltpu.CompilerParams(
            dimension_semantics=("arbitrary",)),
    )(s_feats, q_pts, gathered, s1, s2, W1, b1r, gamr, betr, W2, b2r, kp3)


def kernel(q_pts, s_pts, s_feats, neighb_inds, W1, b1, gamma, beta, W2, b2):
    table = jnp.concatenate(
        [s_feats, s_pts, jnp.zeros((_N, _D - _C - 3), jnp.float32)], axis=1)
    idx = neighb_inds.reshape(-1).astype(jnp.int32)
    gathered = _sc_gather(table, idx)
    b1r = b1.reshape(1, _CR)
    s1, s2 = _tc_stats(s_feats, W1, b1r)
    kp3 = jnp.asarray(np.repeat(_kp_const().T, _G, axis=1))
    out = _tc_main(s_feats, q_pts, gathered, s1, s2, W1, b1r,
                   gamma.reshape(1, _CR), beta.reshape(1, _CR),
                   W2, b2.reshape(1, _K * _G), kp3)
    return out


# speedup vs baseline: 2.0230x; 1.1009x over previous
import functools

import jax
import jax.numpy as jnp
import numpy as np
from jax import lax
from jax.experimental import pallas as pl
from jax.experimental.pallas import tpu as pltpu
from jax.experimental.pallas import tpu_sc as plsc

_N = 10000
_H = 16
_C = 256
_K = 15
_CPG = 8
_G = _C // _CPG
_CR = 64
_SIGMA = 1.2
_BN_EPS = 1e-5

_D = 384
_B = _N * _H
_NW = 32
_RPW = _B // _NW
_CHUNK = 40
_NCHUNK = _RPW // _CHUNK

_BN = 200
_NB = _N // _BN
_R = _BN * _H


def _kp_const():
    rng = np.random.RandomState(42)
    pts = rng.randn(_K, 3)
    pts = pts / (np.linalg.norm(pts, axis=1, keepdims=True) + 1e-9)
    r = rng.rand(_K, 1) ** (1.0 / 3.0)
    pts = pts * r * 1.2
    pts[0, :] = 0.0
    return pts.astype(np.float32)


def _sc_gather(table, idx):
    mesh = plsc.VectorSubcoreMesh(core_axis_name="c", subcore_axis_name="s")

    @functools.partial(
        pl.kernel,
        mesh=mesh,
        out_type=jax.ShapeDtypeStruct((_B, _D), jnp.float32),
        scratch_types=[
            pltpu.VMEM((_CHUNK,), jnp.int32),
            pltpu.VMEM((_CHUNK, _D), jnp.float32),
            pltpu.SemaphoreType.DMA,
        ],
    )
    def k(table_hbm, idx_hbm, out_hbm, idx_v, rows_v, sem):
        wid = lax.axis_index("s") * 2 + lax.axis_index("c")
        base = wid * _RPW

        def body(c, carry):
            off = base + c * _CHUNK
            pltpu.sync_copy(idx_hbm.at[pl.ds(off, _CHUNK)], idx_v)
            pltpu.async_copy(table_hbm.at[idx_v], rows_v, sem).wait()
            pltpu.sync_copy(rows_v, out_hbm.at[pl.ds(off, _CHUNK)])
            return carry

        lax.fori_loop(0, _NCHUNK, body, 0)

    return k(table, idx)


def _stats_body(x_ref, w1_ref, b1_ref, s1_ref, s2_ref):
    h = jnp.dot(x_ref[...], w1_ref[...],
                preferred_element_type=jnp.float32) + b1_ref[...]

    @pl.when(pl.program_id(0) == 0)
    def _():
        s1_ref[...] = jnp.zeros_like(s1_ref)
        s2_ref[...] = jnp.zeros_like(s2_ref)

    s1_ref[...] += jnp.sum(h, axis=0, keepdims=True)
    s2_ref[...] += jnp.sum(h * h, axis=0, keepdims=True)


def _tc_stats(s_feats, W1, b1r):
    return pl.pallas_call(
        _stats_body,
        grid=(_NB,),
        in_specs=[
            pl.BlockSpec((_BN, _C), lambda i: (i, 0)),
            pl.BlockSpec((_C, _CR), lambda i: (0, 0)),
            pl.BlockSpec((1, _CR), lambda i: (0, 0)),
        ],
        out_specs=[
            pl.BlockSpec((1, _CR), lambda i: (0, 0)),
            pl.BlockSpec((1, _CR), lambda i: (0, 0)),
        ],
        out_shape=[
            jax.ShapeDtypeStruct((1, _CR), jnp.float32),
            jax.ShapeDtypeStruct((1, _CR), jnp.float32),
        ],
        compiler_params=pltpu.CompilerParams(
            dimension_semantics=("arbitrary",)),
    )(s_feats, W1, b1r)


def _rep_rows(a, m):
    return jnp.broadcast_to(a[:, None, :], (_BN, _H, m)).reshape(_R, m)


def _main_body(x_ref, q_ref, g_ref, s1_ref, s2_ref, w1_ref, b1_ref,
               gam_ref, bet_ref, w2_ref, b2_ref, kp_ref, o_ref):
    x = x_ref[...]
    h = jnp.dot(x, w1_ref[...], preferred_element_type=jnp.float32)
    h = h + b1_ref[...]
    mean = s1_ref[...] * (1.0 / _N)
    var = s2_ref[...] * (1.0 / _N) - mean * mean
    inv = lax.rsqrt(var + _BN_EPS)
    h = (h - mean) * (inv * gam_ref[...]) + bet_ref[...]
    h = jnp.where(h >= 0.0, h, 0.1 * h)
    cw = jnp.dot(h, w2_ref[...], preferred_element_type=jnp.float32)
    cw = cw + b2_ref[...]

    gb = g_ref[...]
    feats = gb[:, :_C]
    q = q_ref[...]
    m5 = kp_ref[...]

    ax = gb[:, _C:_C + 1] - _rep_rows(q[:, 0:1], 1)
    ay = gb[:, _C + 1:_C + 2] - _rep_rows(q[:, 1:2], 1)
    az = gb[:, _C + 2:_C + 3] - _rep_rows(q[:, 2:3], 1)
    r2 = ax * ax + ay * ay + az * az
    p5 = jnp.concatenate([ax, ay, az, r2, jnp.ones_like(ax)], axis=1)
    d2s = jnp.dot(p5, m5, preferred_element_type=jnp.float32)
    nwx = jnp.maximum(1.0 - jnp.sqrt(jnp.maximum(d2s, 0.0)), 0.0)

    cwr = _rep_rows(cw, _K * _G)
    prod480 = nwx * cwr

    kg_ids = lax.broadcasted_iota(jnp.int32, (_K * _G, _G), 0)
    gg_ids = lax.broadcasted_iota(jnp.int32, (_K * _G, _G), 1)
    et = (kg_ids % _G == gg_ids).astype(jnp.float32)
    agg = jnp.dot(prod480, et, preferred_element_type=jnp.float32)

    g_ids = lax.broadcasted_iota(jnp.int32, (_G, _C), 0)
    c_ids = lax.broadcasted_iota(jnp.int32, (_G, _C), 1)
    expand = (c_ids // _CPG == g_ids).astype(jnp.float32)
    agg_exp = jnp.dot(agg, expand, preferred_element_type=jnp.float32)

    prod = agg_exp * feats
    o_ref[...] = jnp.sum(prod.reshape(_BN, _H, _C), axis=1)


def _tc_main(s_feats, q_pts, gathered, s1, s2, W1, b1r, gamr, betr,
             W2, b2r, kp3):
    return pl.pallas_call(
        _main_body,
        grid=(_NB,),
        in_specs=[
            pl.BlockSpec((_BN, _C), lambda i: (i, 0)),
            pl.BlockSpec((_BN, 3), lambda i: (i, 0)),
            pl.BlockSpec((_R, _D), lambda i: (i, 0)),
            pl.BlockSpec((1, _CR), lambda i: (0, 0)),
            pl.BlockSpec((1, _CR), lambda i: (0, 0)),
            pl.BlockSpec((_C, _CR), lambda i: (0, 0)),
            pl.BlockSpec((1, _CR), lambda i: (0, 0)),
            pl.BlockSpec((1, _CR), lambda i: (0, 0)),
            pl.BlockSpec((1, _CR), lambda i: (0, 0)),
            pl.BlockSpec((_CR, _K * _G), lambda i: (0, 0)),
            pl.BlockSpec((1, _K * _G), lambda i: (0, 0)),
            pl.BlockSpec((5, _K * _G), lambda i: (0, 0)),
        ],
        out_specs=pl.BlockSpec((_BN, _C), lambda i: (i, 0)),
        out_shape=jax.ShapeDtypeStruct((_N, _C), jnp.float32),
        compiler_params=pltpu.CompilerParams(
            dimension_semantics=("arbitrary",)),
    )(s_feats, q_pts, gathered, s1, s2, W1, b1r, gamr, betr, W2, b2r, kp3)


def kernel(q_pts, s_pts, s_feats, neighb_inds, W1, b1, gamma, beta, W2, b2):
    table = jnp.concatenate(
        [s_feats, s_pts, jnp.zeros((_N, _D - _C - 3), jnp.float32)], axis=1)
    idx = neighb_inds.reshape(-1).astype(jnp.int32)
    gathered = _sc_gather(table, idx)
    b1r = b1.reshape(1, _CR)
    s1, s2 = _tc_stats(s_feats, W1, b1r)
    kpr = np.repeat(_kp_const().T, _G, axis=1)
    inv_s2 = 1.0 / (_SIGMA * _SIGMA)
    m5 = np.concatenate([
        -2.0 * inv_s2 * kpr,
        np.full((1, _K * _G), inv_s2, np.float32),
        inv_s2 * np.sum(kpr * kpr, axis=0, keepdims=True),
    ], axis=0).astype(np.float32)
    kp3 = jnp.asarray(m5)
    out = _tc_main(s_feats, q_pts, gathered, s1, s2, W1, b1r,
                   gamma.reshape(1, _CR), beta.reshape(1, _CR),
                   W2, b2.reshape(1, _K * _G), kp3)
    return out
